# Initial kernel scaffold; baseline (speedup 1.0000x reference)
import jax
import jax.numpy as jnp
from jax.experimental import pallas as pl


def kernel(des, tweet, num_prop, cat_prop, new_feature, edge_index, edge_type, W_des, b_des, W_tweet, b_tweet, W_num, b_num, W_cat, b_cat, W_new, b_new, attn_w, W_in, b_in, rel_w, root_w, rgcn_b, W_o1, b_o1, W_o2, b_o2):
    raise NotImplementedError("write your pallas kernel here")


# speedup vs baseline: 5.2523x; 5.2523x over previous
import functools

import jax
import jax.numpy as jnp
from jax import lax
from jax.experimental import pallas as pl
from jax.experimental.pallas import tpu as pltpu
import jax.experimental.pallas.tpu_sc as plsc

N = 10000
E = 320000
NUM_REL = 2
COMMON = 64
EMB = 128
HALF = EMB // 2

NC = 2
NS = 16
CH = 128
OUTER = 16
EPAD = 327680
NCHUNK = EPAD // (NS * CH)
NOUTER = NCHUNK // OUTER
AROWS = 20480
NPAD = 10240
XROWS = NPAD // NS
ZROWS = AROWS // NS
HROWS = AROWS // 128
RBLK = 1000


def _lrelu(z):
    return jnp.where(z >= 0, z, 0.01 * z)


def _make_sc_agg(do_counts):
    a_type = jax.ShapeDtypeStruct((NC, AROWS, HALF), jnp.float32)
    if do_counts:
        out_types = [a_type,
                     jax.ShapeDtypeStruct((AROWS, 16), jnp.float32)]
    else:
        out_types = a_type
    mesh = plsc.VectorSubcoreMesh(core_axis_name="c", subcore_axis_name="s")

    @functools.partial(
        pl.kernel,
        out_type=out_types,
        mesh=mesh,
        compiler_params=pltpu.CompilerParams(use_tc_tiling_on_sc=False),
        scratch_types=[
            pltpu.VMEM((OUTER, CH), jnp.int32),
            pltpu.VMEM((OUTER, CH), jnp.int32),
            pltpu.VMEM((OUTER, CH), jnp.int32),
            pltpu.VMEM((OUTER, CH), jnp.int32),
            pltpu.VMEM((CH, HALF), jnp.float32),
            pltpu.VMEM((CH, 16), jnp.float32),
            pltpu.VMEM_SHARED((AROWS, HALF), jnp.float32),
            pltpu.VMEM_SHARED((AROWS, 16), jnp.float32),
        ],
    )
    def sc_agg(x_lo, x_hi, src2d, dst2d, et2d, z_rows, z_c, ones_in, *rest):
        if do_counts:
            (a_out, cnt_out, src_v, dst_v, et_v, comb_v, rows_v, ones_v,
             acc_sh, c_sh) = rest
        else:
            (a_out, src_v, dst_v, et_v, comb_v, rows_v, ones_v,
             acc_sh, c_sh) = rest
        cid = lax.axis_index("c")
        sid = lax.axis_index("s")

        pltpu.sync_copy(z_rows, acc_sh.at[pl.ds(sid * ZROWS, ZROWS)])
        if do_counts:
            @pl.when(cid == 0)
            def _():
                pltpu.sync_copy(z_c, c_sh.at[pl.ds(sid * ZROWS, ZROWS)])
                pltpu.sync_copy(ones_in, ones_v)
        plsc.subcore_barrier()

        def outer_body(o, _):
            rb = sid * NCHUNK + o * OUTER
            pltpu.sync_copy(src2d.at[pl.ds(rb, OUTER)], src_v)
            pltpu.sync_copy(dst2d.at[pl.ds(rb, OUTER)], dst_v)
            pltpu.sync_copy(et2d.at[pl.ds(rb, OUTER)], et_v)

            def row_body(j, _):
                def grp_body(g, _):
                    dd = dst_v[j, pl.ds(g * 16, 16)]
                    tt = et_v[j, pl.ds(g * 16, 16)]
                    comb_v[j, pl.ds(g * 16, 16)] = dd + N * tt
                    return 0
                lax.fori_loop(0, CH // 16, grp_body, 0)
                @pl.when(cid == 0)
                def _():
                    pltpu.sync_copy(x_lo.at[src_v.at[j]], rows_v)
                @pl.when(cid == 1)
                def _():
                    pltpu.sync_copy(x_hi.at[src_v.at[j]], rows_v)
                pltpu.sync_copy(rows_v, acc_sh.at[comb_v.at[j]], add=True)
                if do_counts:
                    @pl.when(cid == 0)
                    def _():
                        pltpu.sync_copy(ones_v, c_sh.at[comb_v.at[j]],
                                        add=True)
                return 0
            lax.fori_loop(0, OUTER, row_body, 0)
            return 0
        lax.fori_loop(0, NOUTER, outer_body, 0)

        plsc.subcore_barrier()
        pltpu.sync_copy(acc_sh.at[pl.ds(sid * ZROWS, ZROWS)],
                        a_out.at[cid, pl.ds(sid * ZROWS, ZROWS)])
        if do_counts:
            @pl.when(cid == 0)
            def _():
                pltpu.sync_copy(c_sh.at[pl.ds(sid * ZROWS, ZROWS)],
                                cnt_out.at[pl.ds(sid * ZROWS, ZROWS)])

    return sc_agg


_sc_cache = {}


def _get_sc_agg(do_counts):
    if do_counts not in _sc_cache:
        _sc_cache[do_counts] = _make_sc_agg(do_counts)
    return _sc_cache[do_counts]


def _k1_body(des_r, tweet_r, num_r, cat_r, new_r,
             wd_r, bd_r, wt_r, bt_r, wn_r, bn_r, wc_r, bc_r, wnf_r, bnf_r,
             aw_r, win_r, bin_r, out_r):
    d = _lrelu(jnp.dot(des_r[...], wd_r[...],
                       preferred_element_type=jnp.float32) + bd_r[...])
    t = _lrelu(jnp.dot(tweet_r[...], wt_r[...],
                       preferred_element_type=jnp.float32) + bt_r[...])
    n = _lrelu(jnp.dot(num_r[...], wn_r[...],
                       preferred_element_type=jnp.float32) + bn_r[...])
    c = _lrelu(jnp.dot(cat_r[...], wc_r[...],
                       preferred_element_type=jnp.float32) + bc_r[...])
    nf = _lrelu(jnp.dot(new_r[...], wnf_r[...],
                        preferred_element_type=jnp.float32) + bnf_r[...])
    aw = aw_r[...]
    ew = jnp.exp(aw - jnp.max(aw))
    w = ew / jnp.sum(ew)
    fused = (w[0, 0] * d + w[1, 0] * t + w[2, 0] * n
             + w[3, 0] * c + w[4, 0] * nf)
    x = _lrelu(jnp.dot(fused, win_r[...],
                       preferred_element_type=jnp.float32) + bin_r[...])
    out_r[0] = x[:, :HALF]
    out_r[1] = x[:, HALF:]


def _combine(xb_r, a0_r, a1_r, cnt_r, relw_r, rootw_r, rgcnb_r):
    x = jnp.concatenate([xb_r[0], xb_r[1]], axis=1)
    a0 = jnp.concatenate([a0_r[0], a0_r[1]], axis=1)
    a1 = jnp.concatenate([a1_r[0], a1_r[1]], axis=1)
    csum = cnt_r[...]
    c0 = jnp.clip(csum[:, 0:1], 1.0, None)
    c1 = jnp.clip(csum[:, 1:2], 1.0, None)
    out = jnp.dot(x, rootw_r[...], preferred_element_type=jnp.float32)
    out = out + rgcnb_r[...]
    out = out + jnp.dot(a0, relw_r[0], preferred_element_type=jnp.float32) / c0
    out = out + jnp.dot(a1, relw_r[1], preferred_element_type=jnp.float32) / c1
    return out


def _k2_body(xb_r, a0_r, a1_r, cnt_r, relw_r, rootw_r, rgcnb_r, out_r):
    out = _combine(xb_r, a0_r, a1_r, cnt_r, relw_r, rootw_r, rgcnb_r)
    out_r[0] = out[:, :HALF]
    out_r[1] = out[:, HALF:]


def _k3_body(xb_r, a0_r, a1_r, cnt_r, relw_r, rootw_r, rgcnb_r,
             wo1_r, bo1_r, wo2_r, bo2_r, out_r):
    out = _combine(xb_r, a0_r, a1_r, cnt_r, relw_r, rootw_r, rgcnb_r)
    h = _lrelu(jnp.dot(out, wo1_r[...], preferred_element_type=jnp.float32)
               + bo1_r[...])
    out_r[...] = jnp.dot(h, wo2_r[...],
                         preferred_element_type=jnp.float32) + bo2_r[...]


def _full(shape):
    return pl.BlockSpec(shape, lambda i: tuple(0 for _ in shape))


def _rows(shape, dim=0):
    def imap(i):
        idx = [0] * len(shape)
        idx[dim] = i
        return tuple(idx)
    return pl.BlockSpec(shape, imap)


def kernel(des, tweet, num_prop, cat_prop, new_feature, edge_index, edge_type,
           W_des, b_des, W_tweet, b_tweet, W_num, b_num, W_cat, b_cat,
           W_new, b_new, attn_w, W_in, b_in, rel_w, root_w, rgcn_b,
           W_o1, b_o1, W_o2, b_o2):
    grid = (N // RBLK,)

    xb0 = pl.pallas_call(
        _k1_body,
        grid=grid,
        in_specs=[
            _rows((RBLK, 768)), _rows((RBLK, 768)),
            _rows((RBLK, 7)), _rows((RBLK, 3)), _rows((RBLK, 1)),
            _full((768, COMMON)), _full((1, COMMON)),
            _full((768, COMMON)), _full((1, COMMON)),
            _full((7, COMMON)), _full((1, COMMON)),
            _full((3, COMMON)), _full((1, COMMON)),
            _full((1, COMMON)), _full((1, COMMON)),
            _full((5, 1)),
            _full((COMMON, EMB)), _full((1, EMB)),
        ],
        out_specs=_rows((NC, RBLK, HALF), dim=1),
        out_shape=jax.ShapeDtypeStruct((NC, N, HALF), jnp.float32),
    )(des, tweet, num_prop, cat_prop, new_feature,
      W_des, b_des.reshape(1, -1), W_tweet, b_tweet.reshape(1, -1),
      W_num, b_num.reshape(1, -1), W_cat, b_cat.reshape(1, -1),
      W_new, b_new.reshape(1, -1), attn_w, W_in, b_in.reshape(1, -1))

    src = edge_index[0].astype(jnp.int32)
    dst = edge_index[1].astype(jnp.int32)
    et = edge_type.astype(jnp.int32)
    pad = EPAD - E
    src2d = jnp.concatenate([src, jnp.zeros((pad,), jnp.int32)]).reshape(-1, CH)
    dst2d = jnp.concatenate([dst, jnp.zeros((pad,), jnp.int32)]).reshape(-1, CH)
    et2d = jnp.concatenate(
        [et, jnp.full((pad,), NUM_REL, jnp.int32)]).reshape(-1, CH)
    z_rows = jnp.zeros((ZROWS, HALF), jnp.float32)
    z_c = jnp.zeros((ZROWS, 16), jnp.float32)
    ones_in = jnp.ones((CH, 16), jnp.float32)

    a1_out, cnt_out = _get_sc_agg(True)(
        xb0[0], xb0[1], src2d, dst2d, et2d, z_rows, z_c, ones_in)
    cnt = cnt_out[:2 * N, 0].reshape(NUM_REL, N).T

    relw_specs = [
        _full((NC, RBLK, HALF)),
        pl.BlockSpec((NC, RBLK, HALF), lambda i: (0, i, 0)),
        pl.BlockSpec((NC, RBLK, HALF), lambda i: (0, i + N // RBLK, 0)),
        pl.BlockSpec((RBLK, NUM_REL), lambda i: (i, 0)),
        _full((NUM_REL, EMB, EMB)), _full((EMB, EMB)), _full((1, EMB)),
    ]
    relw_specs[0] = pl.BlockSpec((NC, RBLK, HALF), lambda i: (0, i, 0))

    xb1 = pl.pallas_call(
        _k2_body,
        grid=grid,
        in_specs=relw_specs,
        out_specs=pl.BlockSpec((NC, RBLK, HALF), lambda i: (0, i, 0)),
        out_shape=jax.ShapeDtypeStruct((NC, N, HALF), jnp.float32),
    )(xb0, a1_out, a1_out, cnt, rel_w, root_w, rgcn_b.reshape(1, -1))

    a2_out = _get_sc_agg(False)(
        xb1[0], xb1[1], src2d, dst2d, et2d, z_rows, z_c, ones_in)

    out = pl.pallas_call(
        _k3_body,
        grid=grid,
        in_specs=relw_specs + [
            _full((EMB, EMB)), _full((1, EMB)),
            _full((EMB, 2)), _full((1, 2)),
        ],
        out_specs=_rows((RBLK, 2)),
        out_shape=jax.ShapeDtypeStruct((N, 2), jnp.float32),
    )(xb1, a2_out, a2_out, cnt, rel_w, root_w, rgcn_b.reshape(1, -1),
      W_o1, b_o1.reshape(1, -1), W_o2, b_o2.reshape(1, -1))
    return out


# speedup vs baseline: 5.9440x; 1.1317x over previous
import functools

import jax
import jax.numpy as jnp
from jax import lax
from jax.experimental import pallas as pl
from jax.experimental.pallas import tpu as pltpu
import jax.experimental.pallas.tpu_sc as plsc

N = 10000
E = 320000
NUM_REL = 2
COMMON = 64
EMB = 128
HALF = EMB // 2

NC = 2
NS = 16
CH = 128
OUTER = 16
EPAD = 327680
NCHUNK = EPAD // (NS * CH)
NOUTER = NCHUNK // OUTER
AROWS = 20480
NPAD = 10240
XROWS = NPAD // NS
ZROWS = AROWS // NS
HROWS = AROWS // 128
RBLK = 1000


def _lrelu(z):
    return jnp.where(z >= 0, z, 0.01 * z)


def _make_sc_agg(do_counts):
    a_type = jax.ShapeDtypeStruct((NC, AROWS, HALF), jnp.float32)
    if do_counts:
        out_types = [a_type,
                     jax.ShapeDtypeStruct((AROWS, 16), jnp.float32)]
    else:
        out_types = a_type
    mesh = plsc.VectorSubcoreMesh(core_axis_name="c", subcore_axis_name="s")

    @functools.partial(
        pl.kernel,
        out_type=out_types,
        mesh=mesh,
        compiler_params=pltpu.CompilerParams(use_tc_tiling_on_sc=False),
        scratch_types=[
            pltpu.VMEM((OUTER, CH), jnp.int32),
            pltpu.VMEM((OUTER, CH), jnp.int32),
            pltpu.VMEM((OUTER, CH), jnp.int32),
            pltpu.VMEM((OUTER, CH), jnp.int32),
            pltpu.VMEM((CH, HALF), jnp.float32),
            pltpu.VMEM((CH, HALF), jnp.float32),
            pltpu.VMEM((CH, 16), jnp.float32),
            pltpu.VMEM_SHARED((AROWS, HALF), jnp.float32),
            pltpu.VMEM_SHARED((AROWS, 16), jnp.float32),
            pltpu.SemaphoreType.DMA,
            pltpu.SemaphoreType.DMA,
            pltpu.SemaphoreType.DMA,
            pltpu.SemaphoreType.DMA,
            pltpu.SemaphoreType.DMA,
        ],
    )
    def sc_agg(x_lo, x_hi, src2d, dst2d, et2d, z_rows, z_c, ones_in, *rest):
        if do_counts:
            (a_out, cnt_out, src_v, dst_v, et_v, comb_v, rows_a, rows_b,
             ones_v, acc_sh, c_sh, gsem0, gsem1, ssem0, ssem1, csem) = rest
        else:
            (a_out, src_v, dst_v, et_v, comb_v, rows_a, rows_b,
             ones_v, acc_sh, c_sh, gsem0, gsem1, ssem0, ssem1, csem) = rest
        cid = lax.axis_index("c")
        sid = lax.axis_index("s")

        pltpu.sync_copy(z_rows, acc_sh.at[pl.ds(sid * ZROWS, ZROWS)])
        if do_counts:
            @pl.when(cid == 0)
            def _():
                pltpu.sync_copy(z_c, c_sh.at[pl.ds(sid * ZROWS, ZROWS)])
                pltpu.sync_copy(ones_in, ones_v)
        plsc.subcore_barrier()

        bufs = (rows_a, rows_b)
        gsems = (gsem0, gsem1)
        ssems = (ssem0, ssem1)

        def gstart(j, b):
            @pl.when(cid == 0)
            def _():
                pltpu.async_copy(x_lo.at[src_v.at[j]], bufs[b], gsems[b])
            @pl.when(cid == 1)
            def _():
                pltpu.async_copy(x_hi.at[src_v.at[j]], bufs[b], gsems[b])

        def outer_body(o, _):
            rb = sid * NCHUNK + o * OUTER
            pltpu.sync_copy(src2d.at[pl.ds(rb, OUTER)], src_v)
            pltpu.sync_copy(dst2d.at[pl.ds(rb, OUTER)], dst_v)
            pltpu.sync_copy(et2d.at[pl.ds(rb, OUTER)], et_v)

            def row_body(j, _):
                def grp_body(g, _):
                    dd = dst_v[j, pl.ds(g * 16, 16)]
                    tt = et_v[j, pl.ds(g * 16, 16)]
                    comb_v[j, pl.ds(g * 16, 16)] = dd + N * tt
                    return 0
                lax.fori_loop(0, CH // 16, grp_body, 0)
                return 0
            lax.fori_loop(0, OUTER, row_body, 0)

            gstart(0, 0)
            for j in range(OUTER):
                b = j & 1
                pltpu.make_async_copy(x_lo.at[src_v.at[j]], bufs[b],
                                      gsems[b]).wait()
                pltpu.async_copy(bufs[b], acc_sh.at[comb_v.at[j]], ssems[b],
                                 add=True)
                if do_counts:
                    @pl.when(cid == 0)
                    def _():
                        pltpu.async_copy(ones_v, c_sh.at[comb_v.at[j]],
                                         csem, add=True)
                if j + 1 < OUTER:
                    if j >= 1:
                        pltpu.make_async_copy(
                            bufs[1 - b], acc_sh.at[comb_v.at[j - 1]],
                            ssems[1 - b]).wait()
                    gstart(j + 1, 1 - b)
            pltpu.make_async_copy(bufs[0], acc_sh.at[comb_v.at[OUTER - 2]],
                                  ssems[0]).wait()
            pltpu.make_async_copy(bufs[1], acc_sh.at[comb_v.at[OUTER - 1]],
                                  ssems[1]).wait()
            if do_counts:
                @pl.when(cid == 0)
                def _():
                    for j in range(OUTER):
                        pltpu.make_async_copy(ones_v, c_sh.at[comb_v.at[j]],
                                              csem).wait()
            return 0
        lax.fori_loop(0, NOUTER, outer_body, 0)

        plsc.subcore_barrier()
        pltpu.sync_copy(acc_sh.at[pl.ds(sid * ZROWS, ZROWS)],
                        a_out.at[cid, pl.ds(sid * ZROWS, ZROWS)])
        if do_counts:
            @pl.when(cid == 0)
            def _():
                pltpu.sync_copy(c_sh.at[pl.ds(sid * ZROWS, ZROWS)],
                                cnt_out.at[pl.ds(sid * ZROWS, ZROWS)])

    return sc_agg


_sc_cache = {}


def _get_sc_agg(do_counts):
    if do_counts not in _sc_cache:
        _sc_cache[do_counts] = _make_sc_agg(do_counts)
    return _sc_cache[do_counts]


def _k1_body(des_r, tweet_r, num_r, cat_r, new_r,
             wd_r, bd_r, wt_r, bt_r, wn_r, bn_r, wc_r, bc_r, wnf_r, bnf_r,
             aw_r, win_r, bin_r, out_r):
    d = _lrelu(jnp.dot(des_r[...], wd_r[...],
                       preferred_element_type=jnp.float32) + bd_r[...])
    t = _lrelu(jnp.dot(tweet_r[...], wt_r[...],
                       preferred_element_type=jnp.float32) + bt_r[...])
    n = _lrelu(jnp.dot(num_r[...], wn_r[...],
                       preferred_element_type=jnp.float32) + bn_r[...])
    c = _lrelu(jnp.dot(cat_r[...], wc_r[...],
                       preferred_element_type=jnp.float32) + bc_r[...])
    nf = _lrelu(jnp.dot(new_r[...], wnf_r[...],
                        preferred_element_type=jnp.float32) + bnf_r[...])
    aw = aw_r[...]
    ew = jnp.exp(aw - jnp.max(aw))
    w = ew / jnp.sum(ew)
    fused = (w[0, 0] * d + w[1, 0] * t + w[2, 0] * n
             + w[3, 0] * c + w[4, 0] * nf)
    x = _lrelu(jnp.dot(fused, win_r[...],
                       preferred_element_type=jnp.float32) + bin_r[...])
    out_r[0] = x[:, :HALF]
    out_r[1] = x[:, HALF:]


def _combine(xb_r, a0_r, a1_r, cnt_r, relw_r, rootw_r, rgcnb_r):
    x = jnp.concatenate([xb_r[0], xb_r[1]], axis=1)
    a0 = jnp.concatenate([a0_r[0], a0_r[1]], axis=1)
    a1 = jnp.concatenate([a1_r[0], a1_r[1]], axis=1)
    csum = cnt_r[...]
    c0 = jnp.clip(csum[:, 0:1], 1.0, None)
    c1 = jnp.clip(csum[:, 1:2], 1.0, None)
    out = jnp.dot(x, rootw_r[...], preferred_element_type=jnp.float32)
    out = out + rgcnb_r[...]
    out = out + jnp.dot(a0, relw_r[0], preferred_element_type=jnp.float32) / c0
    out = out + jnp.dot(a1, relw_r[1], preferred_element_type=jnp.float32) / c1
    return out


def _k2_body(xb_r, a0_r, a1_r, cnt_r, relw_r, rootw_r, rgcnb_r, out_r):
    out = _combine(xb_r, a0_r, a1_r, cnt_r, relw_r, rootw_r, rgcnb_r)
    out_r[0] = out[:, :HALF]
    out_r[1] = out[:, HALF:]


def _k3_body(xb_r, a0_r, a1_r, cnt_r, relw_r, rootw_r, rgcnb_r,
             wo1_r, bo1_r, wo2_r, bo2_r, out_r):
    out = _combine(xb_r, a0_r, a1_r, cnt_r, relw_r, rootw_r, rgcnb_r)
    h = _lrelu(jnp.dot(out, wo1_r[...], preferred_element_type=jnp.float32)
               + bo1_r[...])
    out_r[...] = jnp.dot(h, wo2_r[...],
                         preferred_element_type=jnp.float32) + bo2_r[...]


def _full(shape):
    return pl.BlockSpec(shape, lambda i: tuple(0 for _ in shape))


def _rows(shape, dim=0):
    def imap(i):
        idx = [0] * len(shape)
        idx[dim] = i
        return tuple(idx)
    return pl.BlockSpec(shape, imap)


def kernel(des, tweet, num_prop, cat_prop, new_feature, edge_index, edge_type,
           W_des, b_des, W_tweet, b_tweet, W_num, b_num, W_cat, b_cat,
           W_new, b_new, attn_w, W_in, b_in, rel_w, root_w, rgcn_b,
           W_o1, b_o1, W_o2, b_o2):
    grid = (N // RBLK,)

    xb0 = pl.pallas_call(
        _k1_body,
        grid=grid,
        in_specs=[
            _rows((RBLK, 768)), _rows((RBLK, 768)),
            _rows((RBLK, 7)), _rows((RBLK, 3)), _rows((RBLK, 1)),
            _full((768, COMMON)), _full((1, COMMON)),
            _full((768, COMMON)), _full((1, COMMON)),
            _full((7, COMMON)), _full((1, COMMON)),
            _full((3, COMMON)), _full((1, COMMON)),
            _full((1, COMMON)), _full((1, COMMON)),
            _full((5, 1)),
            _full((COMMON, EMB)), _full((1, EMB)),
        ],
        out_specs=_rows((NC, RBLK, HALF), dim=1),
        out_shape=jax.ShapeDtypeStruct((NC, N, HALF), jnp.float32),
    )(des, tweet, num_prop, cat_prop, new_feature,
      W_des, b_des.reshape(1, -1), W_tweet, b_tweet.reshape(1, -1),
      W_num, b_num.reshape(1, -1), W_cat, b_cat.reshape(1, -1),
      W_new, b_new.reshape(1, -1), attn_w, W_in, b_in.reshape(1, -1))

    src = edge_index[0].astype(jnp.int32)
    dst = edge_index[1].astype(jnp.int32)
    et = edge_type.astype(jnp.int32)
    pad = EPAD - E
    src2d = jnp.concatenate([src, jnp.zeros((pad,), jnp.int32)]).reshape(-1, CH)
    dst2d = jnp.concatenate([dst, jnp.zeros((pad,), jnp.int32)]).reshape(-1, CH)
    et2d = jnp.concatenate(
        [et, jnp.full((pad,), NUM_REL, jnp.int32)]).reshape(-1, CH)
    z_rows = jnp.zeros((ZROWS, HALF), jnp.float32)
    z_c = jnp.zeros((ZROWS, 16), jnp.float32)
    ones_in = jnp.ones((CH, 16), jnp.float32)

    a1_out, cnt_out = _get_sc_agg(True)(
        xb0[0], xb0[1], src2d, dst2d, et2d, z_rows, z_c, ones_in)
    cnt = cnt_out[:2 * N, 0].reshape(NUM_REL, N).T

    relw_specs = [
        _full((NC, RBLK, HALF)),
        pl.BlockSpec((NC, RBLK, HALF), lambda i: (0, i, 0)),
        pl.BlockSpec((NC, RBLK, HALF), lambda i: (0, i + N // RBLK, 0)),
        pl.BlockSpec((RBLK, NUM_REL), lambda i: (i, 0)),
        _full((NUM_REL, EMB, EMB)), _full((EMB, EMB)), _full((1, EMB)),
    ]
    relw_specs[0] = pl.BlockSpec((NC, RBLK, HALF), lambda i: (0, i, 0))

    xb1 = pl.pallas_call(
        _k2_body,
        grid=grid,
        in_specs=relw_specs,
        out_specs=pl.BlockSpec((NC, RBLK, HALF), lambda i: (0, i, 0)),
        out_shape=jax.ShapeDtypeStruct((NC, N, HALF), jnp.float32),
    )(xb0, a1_out, a1_out, cnt, rel_w, root_w, rgcn_b.reshape(1, -1))

    a2_out = _get_sc_agg(False)(
        xb1[0], xb1[1], src2d, dst2d, et2d, z_rows, z_c, ones_in)

    out = pl.pallas_call(
        _k3_body,
        grid=grid,
        in_specs=relw_specs + [
            _full((EMB, EMB)), _full((1, EMB)),
            _full((EMB, 2)), _full((1, 2)),
        ],
        out_specs=_rows((RBLK, 2)),
        out_shape=jax.ShapeDtypeStruct((N, 2), jnp.float32),
    )(xb1, a2_out, a2_out, cnt, rel_w, root_w, rgcn_b.reshape(1, -1),
      W_o1, b_o1.reshape(1, -1), W_o2, b_o2.reshape(1, -1))
    return out


# speedup vs baseline: 7.9533x; 1.3380x over previous
import functools

import jax
import jax.numpy as jnp
from jax import lax
from jax.experimental import pallas as pl
from jax.experimental.pallas import tpu as pltpu
import jax.experimental.pallas.tpu_sc as plsc

N = 10000
E = 320000
NUM_REL = 2
COMMON = 64
EMB = 128
HALF = EMB // 2

NC = 2
NS = 16
CH = 128
OUTER = 16
EPAD = 327680
NCHUNK = EPAD // (NS * CH)
NOUTER = NCHUNK // OUTER
AROWS = 20480
NPAD = 10240
XROWS = NPAD // NS
ZROWS = AROWS // NS
HROWS = AROWS // 128
RBLK = 2000


def _lrelu(z):
    return jnp.where(z >= 0, z, 0.01 * z)


def _make_sc_agg(do_counts):
    a_type = jax.ShapeDtypeStruct((NC, AROWS, HALF), jnp.bfloat16)
    if do_counts:
        out_types = [a_type,
                     jax.ShapeDtypeStruct((AROWS, 16), jnp.float32)]
    else:
        out_types = a_type
    mesh = plsc.VectorSubcoreMesh(core_axis_name="c", subcore_axis_name="s")

    @functools.partial(
        pl.kernel,
        out_type=out_types,
        mesh=mesh,
        compiler_params=pltpu.CompilerParams(use_tc_tiling_on_sc=False),
        scratch_types=[
            pltpu.VMEM((OUTER, CH), jnp.int32),
            pltpu.VMEM((OUTER, CH), jnp.int32),
            pltpu.VMEM((OUTER, CH), jnp.int32),
            pltpu.VMEM((OUTER, CH), jnp.int32),
            pltpu.VMEM((CH, HALF), jnp.bfloat16),
            pltpu.VMEM((CH, HALF), jnp.bfloat16),
            pltpu.VMEM((CH, 16), jnp.float32),
            pltpu.VMEM_SHARED((AROWS, HALF), jnp.bfloat16),
            pltpu.VMEM_SHARED((AROWS, 16), jnp.float32),
            pltpu.SemaphoreType.DMA,
            pltpu.SemaphoreType.DMA,
            pltpu.SemaphoreType.DMA,
            pltpu.SemaphoreType.DMA,
            pltpu.SemaphoreType.DMA,
        ],
    )
    def sc_agg(x_lo, x_hi, src2d, dst2d, et2d, z_rows, z_c, ones_in, *rest):
        if do_counts:
            (a_out, cnt_out, src_v, dst_v, et_v, comb_v, rows_a, rows_b,
             ones_v, acc_sh, c_sh, gsem0, gsem1, ssem0, ssem1, csem) = rest
        else:
            (a_out, src_v, dst_v, et_v, comb_v, rows_a, rows_b,
             ones_v, acc_sh, c_sh, gsem0, gsem1, ssem0, ssem1, csem) = rest
        cid = lax.axis_index("c")
        sid = lax.axis_index("s")

        pltpu.sync_copy(z_rows, acc_sh.at[pl.ds(sid * ZROWS, ZROWS)])
        if do_counts:
            @pl.when(cid == 0)
            def _():
                pltpu.sync_copy(z_c, c_sh.at[pl.ds(sid * ZROWS, ZROWS)])
                pltpu.sync_copy(ones_in, ones_v)
        plsc.subcore_barrier()

        bufs = (rows_a, rows_b)
        gsems = (gsem0, gsem1)
        ssems = (ssem0, ssem1)

        def gstart(j, b):
            @pl.when(cid == 0)
            def _():
                pltpu.async_copy(x_lo.at[src_v.at[j]], bufs[b], gsems[b])
            @pl.when(cid == 1)
            def _():
                pltpu.async_copy(x_hi.at[src_v.at[j]], bufs[b], gsems[b])

        def outer_body(o, _):
            rb = sid * NCHUNK + o * OUTER
            pltpu.sync_copy(src2d.at[pl.ds(rb, OUTER)], src_v)
            pltpu.sync_copy(dst2d.at[pl.ds(rb, OUTER)], dst_v)
            pltpu.sync_copy(et2d.at[pl.ds(rb, OUTER)], et_v)

            def row_body(j, _):
                def grp_body(g, _):
                    dd = dst_v[j, pl.ds(g * 16, 16)]
                    tt = et_v[j, pl.ds(g * 16, 16)]
                    comb_v[j, pl.ds(g * 16, 16)] = dd + N * tt
                    return 0
                lax.fori_loop(0, CH // 16, grp_body, 0)
                return 0
            lax.fori_loop(0, OUTER, row_body, 0)

            gstart(0, 0)
            for j in range(OUTER):
                b = j & 1
                pltpu.make_async_copy(x_lo.at[src_v.at[j]], bufs[b],
                                      gsems[b]).wait()
                pltpu.async_copy(bufs[b], acc_sh.at[comb_v.at[j]], ssems[b],
                                 add=True)
                if do_counts:
                    @pl.when(cid == 0)
                    def _():
                        pltpu.async_copy(ones_v, c_sh.at[comb_v.at[j]],
                                         csem, add=True)
                if j + 1 < OUTER:
                    if j >= 1:
                        pltpu.make_async_copy(
                            bufs[1 - b], acc_sh.at[comb_v.at[j - 1]],
                            ssems[1 - b]).wait()
                    gstart(j + 1, 1 - b)
            pltpu.make_async_copy(bufs[0], acc_sh.at[comb_v.at[OUTER - 2]],
                                  ssems[0]).wait()
            pltpu.make_async_copy(bufs[1], acc_sh.at[comb_v.at[OUTER - 1]],
                                  ssems[1]).wait()
            if do_counts:
                @pl.when(cid == 0)
                def _():
                    for j in range(OUTER):
                        pltpu.make_async_copy(ones_v, c_sh.at[comb_v.at[j]],
                                              csem).wait()
            return 0
        lax.fori_loop(0, NOUTER, outer_body, 0)

        plsc.subcore_barrier()
        pltpu.sync_copy(acc_sh.at[pl.ds(sid * ZROWS, ZROWS)],
                        a_out.at[cid, pl.ds(sid * ZROWS, ZROWS)])
        if do_counts:
            @pl.when(cid == 0)
            def _():
                pltpu.sync_copy(c_sh.at[pl.ds(sid * ZROWS, ZROWS)],
                                cnt_out.at[pl.ds(sid * ZROWS, ZROWS)])

    return sc_agg


_sc_cache = {}


def _get_sc_agg(do_counts):
    if do_counts not in _sc_cache:
        _sc_cache[do_counts] = _make_sc_agg(do_counts)
    return _sc_cache[do_counts]


def _k1_body(des_r, tweet_r, num_r, cat_r, new_r,
             wd_r, bd_r, wt_r, bt_r, wn_r, bn_r, wc_r, bc_r, wnf_r, bnf_r,
             aw_r, win_r, bin_r, out_r, outb_r):
    d = _lrelu(jnp.dot(des_r[...], wd_r[...],
                       preferred_element_type=jnp.float32) + bd_r[...])
    t = _lrelu(jnp.dot(tweet_r[...], wt_r[...],
                       preferred_element_type=jnp.float32) + bt_r[...])
    n = _lrelu(jnp.dot(num_r[...], wn_r[...],
                       preferred_element_type=jnp.float32) + bn_r[...])
    c = _lrelu(jnp.dot(cat_r[...], wc_r[...],
                       preferred_element_type=jnp.float32) + bc_r[...])
    nf = _lrelu(jnp.dot(new_r[...], wnf_r[...],
                        preferred_element_type=jnp.float32) + bnf_r[...])
    aw = aw_r[...]
    ew = jnp.exp(aw - jnp.max(aw))
    w = ew / jnp.sum(ew)
    fused = (w[0, 0] * d + w[1, 0] * t + w[2, 0] * n
             + w[3, 0] * c + w[4, 0] * nf)
    x = _lrelu(jnp.dot(fused, win_r[...],
                       preferred_element_type=jnp.float32) + bin_r[...])
    out_r[0] = x[:, :HALF]
    out_r[1] = x[:, HALF:]
    outb_r[0] = x[:, :HALF].astype(jnp.bfloat16)
    outb_r[1] = x[:, HALF:].astype(jnp.bfloat16)


def _combine(xb_r, a0_r, a1_r, cnt_r, relw_r, rootw_r, rgcnb_r):
    x = jnp.concatenate([xb_r[0], xb_r[1]], axis=1)
    a0 = jnp.concatenate([a0_r[0], a0_r[1]], axis=1).astype(jnp.float32)
    a1 = jnp.concatenate([a1_r[0], a1_r[1]], axis=1).astype(jnp.float32)
    csum = cnt_r[...]
    c0 = jnp.clip(csum[:, 0:1], 1.0, None)
    c1 = jnp.clip(csum[:, 1:2], 1.0, None)
    out = jnp.dot(x, rootw_r[...], preferred_element_type=jnp.float32)
    out = out + rgcnb_r[...]
    out = out + jnp.dot(a0, relw_r[0], preferred_element_type=jnp.float32) / c0
    out = out + jnp.dot(a1, relw_r[1], preferred_element_type=jnp.float32) / c1
    return out


def _k2_body(xb_r, a0_r, a1_r, cnt_r, relw_r, rootw_r, rgcnb_r, out_r, outb_r):
    out = _combine(xb_r, a0_r, a1_r, cnt_r, relw_r, rootw_r, rgcnb_r)
    out_r[0] = out[:, :HALF]
    out_r[1] = out[:, HALF:]
    outb_r[0] = out[:, :HALF].astype(jnp.bfloat16)
    outb_r[1] = out[:, HALF:].astype(jnp.bfloat16)


def _k3_body(xb_r, a0_r, a1_r, cnt_r, relw_r, rootw_r, rgcnb_r,
             wo1_r, bo1_r, wo2_r, bo2_r, out_r):
    out = _combine(xb_r, a0_r, a1_r, cnt_r, relw_r, rootw_r, rgcnb_r)
    h = _lrelu(jnp.dot(out, wo1_r[...], preferred_element_type=jnp.float32)
               + bo1_r[...])
    out_r[...] = jnp.dot(h, wo2_r[...],
                         preferred_element_type=jnp.float32) + bo2_r[...]


def _full(shape):
    return pl.BlockSpec(shape, lambda i: tuple(0 for _ in shape))


def _rows(shape, dim=0):
    def imap(i):
        idx = [0] * len(shape)
        idx[dim] = i
        return tuple(idx)
    return pl.BlockSpec(shape, imap)


def kernel(des, tweet, num_prop, cat_prop, new_feature, edge_index, edge_type,
           W_des, b_des, W_tweet, b_tweet, W_num, b_num, W_cat, b_cat,
           W_new, b_new, attn_w, W_in, b_in, rel_w, root_w, rgcn_b,
           W_o1, b_o1, W_o2, b_o2):
    grid = (N // RBLK,)

    xb0, xb0h = pl.pallas_call(
        _k1_body,
        grid=grid,
        in_specs=[
            _rows((RBLK, 768)), _rows((RBLK, 768)),
            _rows((RBLK, 7)), _rows((RBLK, 3)), _rows((RBLK, 1)),
            _full((768, COMMON)), _full((1, COMMON)),
            _full((768, COMMON)), _full((1, COMMON)),
            _full((7, COMMON)), _full((1, COMMON)),
            _full((3, COMMON)), _full((1, COMMON)),
            _full((1, COMMON)), _full((1, COMMON)),
            _full((5, 1)),
            _full((COMMON, EMB)), _full((1, EMB)),
        ],
        out_specs=[_rows((NC, RBLK, HALF), dim=1),
                   _rows((NC, RBLK, HALF), dim=1)],
        out_shape=[jax.ShapeDtypeStruct((NC, N, HALF), jnp.float32),
                   jax.ShapeDtypeStruct((NC, N, HALF), jnp.bfloat16)],
    )(des, tweet, num_prop, cat_prop, new_feature,
      W_des, b_des.reshape(1, -1), W_tweet, b_tweet.reshape(1, -1),
      W_num, b_num.reshape(1, -1), W_cat, b_cat.reshape(1, -1),
      W_new, b_new.reshape(1, -1), attn_w, W_in, b_in.reshape(1, -1))

    src = edge_index[0].astype(jnp.int32)
    dst = edge_index[1].astype(jnp.int32)
    et = edge_type.astype(jnp.int32)
    pad = EPAD - E
    src2d = jnp.concatenate([src, jnp.zeros((pad,), jnp.int32)]).reshape(-1, CH)
    dst2d = jnp.concatenate([dst, jnp.zeros((pad,), jnp.int32)]).reshape(-1, CH)
    et2d = jnp.concatenate(
        [et, jnp.full((pad,), NUM_REL, jnp.int32)]).reshape(-1, CH)
    z_rows = jnp.zeros((ZROWS, HALF), jnp.bfloat16)
    z_c = jnp.zeros((ZROWS, 16), jnp.float32)
    ones_in = jnp.ones((CH, 16), jnp.float32)

    a1_out, cnt_out = _get_sc_agg(True)(
        xb0h[0], xb0h[1], src2d, dst2d, et2d, z_rows, z_c, ones_in)
    cnt = cnt_out[:2 * N, 0].reshape(NUM_REL, N).T

    relw_specs = [
        _full((NC, RBLK, HALF)),
        pl.BlockSpec((NC, RBLK, HALF), lambda i: (0, i, 0)),
        pl.BlockSpec((NC, RBLK, HALF), lambda i: (0, i + N // RBLK, 0)),
        pl.BlockSpec((RBLK, NUM_REL), lambda i: (i, 0)),
        _full((NUM_REL, EMB, EMB)), _full((EMB, EMB)), _full((1, EMB)),
    ]
    relw_specs[0] = pl.BlockSpec((NC, RBLK, HALF), lambda i: (0, i, 0))

    xb1, xb1h = pl.pallas_call(
        _k2_body,
        grid=grid,
        in_specs=relw_specs,
        out_specs=[pl.BlockSpec((NC, RBLK, HALF), lambda i: (0, i, 0)),
                   pl.BlockSpec((NC, RBLK, HALF), lambda i: (0, i, 0))],
        out_shape=[jax.ShapeDtypeStruct((NC, N, HALF), jnp.float32),
                   jax.ShapeDtypeStruct((NC, N, HALF), jnp.bfloat16)],
    )(xb0, a1_out, a1_out, cnt, rel_w, root_w, rgcn_b.reshape(1, -1))

    a2_out = _get_sc_agg(False)(
        xb1h[0], xb1h[1], src2d, dst2d, et2d, z_rows, z_c, ones_in)

    out = pl.pallas_call(
        _k3_body,
        grid=grid,
        in_specs=relw_specs + [
            _full((EMB, EMB)), _full((1, EMB)),
            _full((EMB, 2)), _full((1, 2)),
        ],
        out_specs=_rows((RBLK, 2)),
        out_shape=jax.ShapeDtypeStruct((N, 2), jnp.float32),
    )(xb1, a2_out, a2_out, cnt, rel_w, root_w, rgcn_b.reshape(1, -1),
      W_o1, b_o1.reshape(1, -1), W_o2, b_o2.reshape(1, -1))
    return out


# speedup vs baseline: 8.2484x; 1.0371x over previous
import functools

import jax
import jax.numpy as jnp
from jax import lax
from jax.experimental import pallas as pl
from jax.experimental.pallas import tpu as pltpu
import jax.experimental.pallas.tpu_sc as plsc

N = 10000
E = 320000
NUM_REL = 2
COMMON = 64
EMB = 128
HALF = EMB // 2

NC = 2
NS = 16
CH = 128
OUTER = 16
EPAD = 327680
NCHUNK = EPAD // (NS * CH)
NOUTER = NCHUNK // OUTER
AROWS = 20480
NPAD = 10240
XROWS = NPAD // NS
ZROWS = AROWS // NS
HROWS = AROWS // 128
RBLK = 2000


def _lrelu(z):
    return jnp.where(z >= 0, z, 0.01 * z)


def _make_sc_agg(do_counts):
    a_type = jax.ShapeDtypeStruct((NC, AROWS, HALF), jnp.bfloat16)
    if do_counts:
        out_types = [a_type,
                     jax.ShapeDtypeStruct((AROWS, 16), jnp.float32)]
    else:
        out_types = a_type
    mesh = plsc.VectorSubcoreMesh(core_axis_name="c", subcore_axis_name="s")

    @functools.partial(
        pl.kernel,
        out_type=out_types,
        mesh=mesh,
        compiler_params=pltpu.CompilerParams(use_tc_tiling_on_sc=False),
        scratch_types=[
            pltpu.VMEM((2, OUTER, CH), jnp.int32),
            pltpu.VMEM((2, OUTER, CH), jnp.int32),
            pltpu.VMEM((CH, HALF), jnp.bfloat16),
            pltpu.VMEM((CH, HALF), jnp.bfloat16),
            pltpu.VMEM((CH, 16), jnp.float32),
            pltpu.VMEM_SHARED((AROWS, HALF), jnp.bfloat16),
            pltpu.VMEM_SHARED((AROWS, 16), jnp.float32),
            pltpu.SemaphoreType.DMA,
            pltpu.SemaphoreType.DMA,
            pltpu.SemaphoreType.DMA,
            pltpu.SemaphoreType.DMA,
            pltpu.SemaphoreType.DMA,
            pltpu.SemaphoreType.DMA,
        ],
    )
    def sc_agg(x_lo, x_hi, src2d, comb2d, z_rows, z_c, ones_in, *rest):
        if do_counts:
            (a_out, cnt_out, src_v, comb_v, rows_a, rows_b, ones_v,
             acc_sh, c_sh, gsem0, gsem1, ssem0, ssem1, csem, isem) = rest
        else:
            (a_out, src_v, comb_v, rows_a, rows_b, ones_v,
             acc_sh, c_sh, gsem0, gsem1, ssem0, ssem1, csem, isem) = rest
        cid = lax.axis_index("c")
        sid = lax.axis_index("s")

        pltpu.sync_copy(z_rows, acc_sh.at[pl.ds(sid * ZROWS, ZROWS)])
        if do_counts:
            @pl.when(cid == 0)
            def _():
                pltpu.sync_copy(z_c, c_sh.at[pl.ds(sid * ZROWS, ZROWS)])
                pltpu.sync_copy(ones_in, ones_v)
        plsc.subcore_barrier()

        bufs = (rows_a, rows_b)
        gsems = (gsem0, gsem1)
        ssems = (ssem0, ssem1)

        def idx_start(o, bi):
            rb = sid * NCHUNK + o * OUTER
            pltpu.async_copy(src2d.at[pl.ds(rb, OUTER)], src_v.at[bi], isem)
            pltpu.async_copy(comb2d.at[pl.ds(rb, OUTER)], comb_v.at[bi], isem)

        def idx_wait():
            pltpu.make_async_copy(src2d.at[pl.ds(0, OUTER)], src_v.at[0],
                                  isem).wait()
            pltpu.make_async_copy(comb2d.at[pl.ds(0, OUTER)], comb_v.at[0],
                                  isem).wait()

        idx_start(0, 0)
        idx_wait()

        def outer_body(o, _):
            bi = lax.rem(o, 2)
            @pl.when(o + 1 < NOUTER)
            def _():
                idx_start(o + 1, 1 - bi)

            def gstart(j, b):
                @pl.when(cid == 0)
                def _():
                    pltpu.async_copy(x_lo.at[src_v.at[bi, j]], bufs[b],
                                     gsems[b])
                @pl.when(cid == 1)
                def _():
                    pltpu.async_copy(x_hi.at[src_v.at[bi, j]], bufs[b],
                                     gsems[b])

            gstart(0, 0)
            for j in range(OUTER):
                b = j & 1
                pltpu.make_async_copy(x_lo.at[src_v.at[bi, j]], bufs[b],
                                      gsems[b]).wait()
                pltpu.async_copy(bufs[b], acc_sh.at[comb_v.at[bi, j]],
                                 ssems[b], add=True)
                if do_counts:
                    @pl.when(cid == 0)
                    def _():
                        pltpu.async_copy(ones_v, c_sh.at[comb_v.at[bi, j]],
                                         csem, add=True)
                if j + 1 < OUTER:
                    if j >= 1:
                        pltpu.make_async_copy(
                            bufs[1 - b], acc_sh.at[comb_v.at[bi, j - 1]],
                            ssems[1 - b]).wait()
                    gstart(j + 1, 1 - b)
            pltpu.make_async_copy(bufs[0], acc_sh.at[comb_v.at[bi, OUTER - 2]],
                                  ssems[0]).wait()
            pltpu.make_async_copy(bufs[1], acc_sh.at[comb_v.at[bi, OUTER - 1]],
                                  ssems[1]).wait()
            if do_counts:
                @pl.when(cid == 0)
                def _():
                    for j in range(OUTER):
                        pltpu.make_async_copy(
                            ones_v, c_sh.at[comb_v.at[bi, j]], csem).wait()
            @pl.when(o + 1 < NOUTER)
            def _():
                idx_wait()
            return 0
        lax.fori_loop(0, NOUTER, outer_body, 0)

        plsc.subcore_barrier()
        pltpu.sync_copy(acc_sh.at[pl.ds(sid * ZROWS, ZROWS)],
                        a_out.at[cid, pl.ds(sid * ZROWS, ZROWS)])
        if do_counts:
            @pl.when(cid == 0)
            def _():
                pltpu.sync_copy(c_sh.at[pl.ds(sid * ZROWS, ZROWS)],
                                cnt_out.at[pl.ds(sid * ZROWS, ZROWS)])

    return sc_agg


_sc_cache = {}


def _get_sc_agg(do_counts):
    if do_counts not in _sc_cache:
        _sc_cache[do_counts] = _make_sc_agg(do_counts)
    return _sc_cache[do_counts]


def _comb_body(d_r, e_r, o_r):
    o_r[...] = d_r[...] + N * e_r[...]


def _k1_body(des_r, tweet_r, num_r, cat_r, new_r,
             wd_r, bd_r, wt_r, bt_r, wn_r, bn_r, wc_r, bc_r, wnf_r, bnf_r,
             aw_r, win_r, bin_r, out_r, outb_r):
    d = _lrelu(jnp.dot(des_r[...], wd_r[...],
                       preferred_element_type=jnp.float32) + bd_r[...])
    t = _lrelu(jnp.dot(tweet_r[...], wt_r[...],
                       preferred_element_type=jnp.float32) + bt_r[...])
    n = _lrelu(jnp.dot(num_r[...], wn_r[...],
                       preferred_element_type=jnp.float32) + bn_r[...])
    c = _lrelu(jnp.dot(cat_r[...], wc_r[...],
                       preferred_element_type=jnp.float32) + bc_r[...])
    nf = _lrelu(jnp.dot(new_r[...], wnf_r[...],
                        preferred_element_type=jnp.float32) + bnf_r[...])
    aw = aw_r[...]
    ew = jnp.exp(aw - jnp.max(aw))
    w = ew / jnp.sum(ew)
    fused = (w[0, 0] * d + w[1, 0] * t + w[2, 0] * n
             + w[3, 0] * c + w[4, 0] * nf)
    x = _lrelu(jnp.dot(fused, win_r[...],
                       preferred_element_type=jnp.float32) + bin_r[...])
    out_r[0] = x[:, :HALF]
    out_r[1] = x[:, HALF:]
    outb_r[0] = x[:, :HALF].astype(jnp.bfloat16)
    outb_r[1] = x[:, HALF:].astype(jnp.bfloat16)


def _combine(xb_r, a0_r, a1_r, cnt_r, relw_r, rootw_r, rgcnb_r):
    x = jnp.concatenate([xb_r[0], xb_r[1]], axis=1)
    a0 = jnp.concatenate([a0_r[0], a0_r[1]], axis=1).astype(jnp.float32)
    a1 = jnp.concatenate([a1_r[0], a1_r[1]], axis=1).astype(jnp.float32)
    csum = cnt_r[...]
    c0 = jnp.clip(csum[:, 0:1], 1.0, None)
    c1 = jnp.clip(csum[:, 1:2], 1.0, None)
    out = jnp.dot(x, rootw_r[...], preferred_element_type=jnp.float32)
    out = out + rgcnb_r[...]
    out = out + jnp.dot(a0, relw_r[0], preferred_element_type=jnp.float32) / c0
    out = out + jnp.dot(a1, relw_r[1], preferred_element_type=jnp.float32) / c1
    return out


def _k2_body(xb_r, a0_r, a1_r, cnt_r, relw_r, rootw_r, rgcnb_r, out_r, outb_r):
    out = _combine(xb_r, a0_r, a1_r, cnt_r, relw_r, rootw_r, rgcnb_r)
    out_r[0] = out[:, :HALF]
    out_r[1] = out[:, HALF:]
    outb_r[0] = out[:, :HALF].astype(jnp.bfloat16)
    outb_r[1] = out[:, HALF:].astype(jnp.bfloat16)


def _k3_body(xb_r, a0_r, a1_r, cnt_r, relw_r, rootw_r, rgcnb_r,
             wo1_r, bo1_r, wo2_r, bo2_r, out_r):
    out = _combine(xb_r, a0_r, a1_r, cnt_r, relw_r, rootw_r, rgcnb_r)
    h = _lrelu(jnp.dot(out, wo1_r[...], preferred_element_type=jnp.float32)
               + bo1_r[...])
    out_r[...] = jnp.dot(h, wo2_r[...],
                         preferred_element_type=jnp.float32) + bo2_r[...]


def _full(shape):
    return pl.BlockSpec(shape, lambda i: tuple(0 for _ in shape))


def _rows(shape, dim=0):
    def imap(i):
        idx = [0] * len(shape)
        idx[dim] = i
        return tuple(idx)
    return pl.BlockSpec(shape, imap)


def kernel(des, tweet, num_prop, cat_prop, new_feature, edge_index, edge_type,
           W_des, b_des, W_tweet, b_tweet, W_num, b_num, W_cat, b_cat,
           W_new, b_new, attn_w, W_in, b_in, rel_w, root_w, rgcn_b,
           W_o1, b_o1, W_o2, b_o2):
    grid = (N // RBLK,)

    xb0, xb0h = pl.pallas_call(
        _k1_body,
        grid=grid,
        in_specs=[
            _rows((RBLK, 768)), _rows((RBLK, 768)),
            _rows((RBLK, 7)), _rows((RBLK, 3)), _rows((RBLK, 1)),
            _full((768, COMMON)), _full((1, COMMON)),
            _full((768, COMMON)), _full((1, COMMON)),
            _full((7, COMMON)), _full((1, COMMON)),
            _full((3, COMMON)), _full((1, COMMON)),
            _full((1, COMMON)), _full((1, COMMON)),
            _full((5, 1)),
            _full((COMMON, EMB)), _full((1, EMB)),
        ],
        out_specs=[_rows((NC, RBLK, HALF), dim=1),
                   _rows((NC, RBLK, HALF), dim=1)],
        out_shape=[jax.ShapeDtypeStruct((NC, N, HALF), jnp.float32),
                   jax.ShapeDtypeStruct((NC, N, HALF), jnp.bfloat16)],
    )(des, tweet, num_prop, cat_prop, new_feature,
      W_des, b_des.reshape(1, -1), W_tweet, b_tweet.reshape(1, -1),
      W_num, b_num.reshape(1, -1), W_cat, b_cat.reshape(1, -1),
      W_new, b_new.reshape(1, -1), attn_w, W_in, b_in.reshape(1, -1))

    src = edge_index[0].astype(jnp.int32)
    dst = edge_index[1].astype(jnp.int32)
    et = edge_type.astype(jnp.int32)
    pad = EPAD - E
    src2d = jnp.concatenate([src, jnp.zeros((pad,), jnp.int32)]).reshape(-1, CH)
    dst2d = jnp.concatenate([dst, jnp.zeros((pad,), jnp.int32)]).reshape(-1, CH)
    et2d = jnp.concatenate(
        [et, jnp.full((pad,), NUM_REL, jnp.int32)]).reshape(-1, CH)
    z_rows = jnp.zeros((ZROWS, HALF), jnp.bfloat16)
    z_c = jnp.zeros((ZROWS, 16), jnp.float32)
    ones_in = jnp.ones((CH, 16), jnp.float32)

    comb2d = pl.pallas_call(
        _comb_body,
        out_shape=jax.ShapeDtypeStruct((EPAD // CH, CH), jnp.int32),
    )(dst2d, et2d)

    a1_out, cnt_out = _get_sc_agg(True)(
        xb0h[0], xb0h[1], src2d, comb2d, z_rows, z_c, ones_in)
    cnt = cnt_out[:2 * N, 0].reshape(NUM_REL, N).T

    relw_specs = [
        _full((NC, RBLK, HALF)),
        pl.BlockSpec((NC, RBLK, HALF), lambda i: (0, i, 0)),
        pl.BlockSpec((NC, RBLK, HALF), lambda i: (0, i + N // RBLK, 0)),
        pl.BlockSpec((RBLK, NUM_REL), lambda i: (i, 0)),
        _full((NUM_REL, EMB, EMB)), _full((EMB, EMB)), _full((1, EMB)),
    ]
    relw_specs[0] = pl.BlockSpec((NC, RBLK, HALF), lambda i: (0, i, 0))

    xb1, xb1h = pl.pallas_call(
        _k2_body,
        grid=grid,
        in_specs=relw_specs,
        out_specs=[pl.BlockSpec((NC, RBLK, HALF), lambda i: (0, i, 0)),
                   pl.BlockSpec((NC, RBLK, HALF), lambda i: (0, i, 0))],
        out_shape=[jax.ShapeDtypeStruct((NC, N, HALF), jnp.float32),
                   jax.ShapeDtypeStruct((NC, N, HALF), jnp.bfloat16)],
    )(xb0, a1_out, a1_out, cnt, rel_w, root_w, rgcn_b.reshape(1, -1))

    a2_out = _get_sc_agg(False)(
        xb1h[0], xb1h[1], src2d, comb2d, z_rows, z_c, ones_in)

    out = pl.pallas_call(
        _k3_body,
        grid=grid,
        in_specs=relw_specs + [
            _full((EMB, EMB)), _full((1, EMB)),
            _full((EMB, 2)), _full((1, 2)),
        ],
        out_specs=_rows((RBLK, 2)),
        out_shape=jax.ShapeDtypeStruct((N, 2), jnp.float32),
    )(xb1, a2_out, a2_out, cnt, rel_w, root_w, rgcn_b.reshape(1, -1),
      W_o1, b_o1.reshape(1, -1), W_o2, b_o2.reshape(1, -1))
    return out
